# baseline (speedup 1.0000x reference)
import functools

import jax
import jax.numpy as jnp
from jax import lax
from jax.experimental import pallas as pl
from jax.experimental.pallas import tpu as pltpu
from jax.experimental.pallas import tpu_sc as plsc

B = 32
D = 512
V = 50257

_ROWS_PER_WORKER = 8
_ACTIVE_WORKERS = B // _ROWS_PER_WORKER

_sc_mesh = plsc.VectorSubcoreMesh(core_axis_name="c", subcore_axis_name="s")


@functools.partial(
    pl.kernel,
    out_type=jax.ShapeDtypeStruct((B, D), jnp.float32),
    mesh=_sc_mesh,
    scratch_types=[
        pltpu.VMEM((_ROWS_PER_WORKER,), jnp.int32),
        pltpu.VMEM((_ROWS_PER_WORKER, D), jnp.float32),
        pltpu.SemaphoreType.DMA,
    ],
)
def _sc_gather(emb_hbm, ids_hbm, out_hbm, idx_v, rows_v, sem):
    info = plsc.get_sparse_core_info()
    nc = info.num_cores
    wid = lax.axis_index("s") * nc + lax.axis_index("c")

    @pl.when(wid < _ACTIVE_WORKERS)
    def _():
        base = wid * _ROWS_PER_WORKER
        pltpu.sync_copy(ids_hbm.at[pl.ds(base, _ROWS_PER_WORKER)], idx_v)
        pltpu.async_copy(emb_hbm.at[idx_v], rows_v, sem).wait()
        pltpu.sync_copy(rows_v, out_hbm.at[pl.ds(base, _ROWS_PER_WORKER)])


_DB = 64
_ND = D // _DB


def _mm_body(xT_ref, b_ref, w_ref, o_ref):
    d = pl.program_id(0)
    acc = lax.dot_general(
        xT_ref[...],
        w_ref[...],
        dimension_numbers=(((0,), (0,)), ((), ())),
        preferred_element_type=jnp.float32,
    )

    @pl.when(d == 0)
    def _():
        o_ref[...] = acc + b_ref[...]

    @pl.when(d != 0)
    def _():
        o_ref[...] += acc


def _tc_project(xT, W, b2d):
    return pl.pallas_call(
        _mm_body,
        grid=(_ND,),
        in_specs=[
            pl.BlockSpec((_DB, B), lambda d: (d, 0)),
            pl.BlockSpec((1, V), lambda d: (0, 0)),
            pl.BlockSpec((_DB, V), lambda d: (d, 0)),
        ],
        out_specs=pl.BlockSpec((B, V), lambda d: (0, 0)),
        out_shape=jax.ShapeDtypeStruct((B, V), jnp.float32),
        compiler_params=pltpu.CompilerParams(
            dimension_semantics=("arbitrary",),
        ),
    )(xT, b2d, W)


def kernel(input_ids, emb, W, b):
    ids = input_ids.reshape(B).astype(jnp.int32)
    xT = _sc_gather(emb, ids).T
    logits = _tc_project(xT, W, b.reshape(1, V))
    return logits.reshape(B, 1, V)


# speedup vs baseline: 2.1826x; 2.1826x over previous
import functools

import jax
import jax.numpy as jnp
from jax import lax
from jax.experimental import pallas as pl
from jax.experimental.pallas import tpu as pltpu
from jax.experimental.pallas import tpu_sc as plsc

B = 32
D = 512
V = 50257

_ROWS_PER_WORKER = 8
_ACTIVE_WORKERS = B // _ROWS_PER_WORKER

_sc_mesh = plsc.VectorSubcoreMesh(core_axis_name="c", subcore_axis_name="s")


@functools.partial(
    pl.kernel,
    out_type=jax.ShapeDtypeStruct((B, D), jnp.float32),
    mesh=_sc_mesh,
    scratch_types=[
        pltpu.VMEM((_ROWS_PER_WORKER,), jnp.int32),
        pltpu.VMEM((_ROWS_PER_WORKER, D), jnp.float32),
        pltpu.SemaphoreType.DMA,
    ],
)
def _sc_gather(emb_hbm, ids_hbm, out_hbm, idx_v, rows_v, sem):
    info = plsc.get_sparse_core_info()
    nc = info.num_cores
    wid = lax.axis_index("s") * nc + lax.axis_index("c")

    @pl.when(wid < _ACTIVE_WORKERS)
    def _():
        base = wid * _ROWS_PER_WORKER
        pltpu.sync_copy(ids_hbm.at[pl.ds(base, _ROWS_PER_WORKER)], idx_v)
        pltpu.async_copy(emb_hbm.at[idx_v], rows_v, sem).wait()
        pltpu.sync_copy(rows_v, out_hbm.at[pl.ds(base, _ROWS_PER_WORKER)])


_VB = 2048
_NV = (V + _VB - 1) // _VB


def _mm_body(x_ref, wt_ref, b_ref, o_ref):
    o_ref[...] = (
        lax.dot_general(
            x_ref[...],
            wt_ref[...],
            dimension_numbers=(((1,), (1,)), ((), ())),
            preferred_element_type=jnp.float32,
        )
        + b_ref[...]
    )


def _tc_project(x, WT, b2d):
    return pl.pallas_call(
        _mm_body,
        grid=(_NV,),
        in_specs=[
            pl.BlockSpec((B, D), lambda v: (0, 0)),
            pl.BlockSpec((_VB, D), lambda v: (v, 0)),
            pl.BlockSpec((1, _VB), lambda v: (0, v)),
        ],
        out_specs=pl.BlockSpec((B, _VB), lambda v: (0, v)),
        out_shape=jax.ShapeDtypeStruct((B, V), jnp.float32),
        compiler_params=pltpu.CompilerParams(
            dimension_semantics=("arbitrary",),
        ),
    )(x, WT, b2d)


def kernel(input_ids, emb, W, b):
    ids = input_ids.reshape(B).astype(jnp.int32)
    x = _sc_gather(emb, ids)
    logits = _tc_project(x, W.T, b.reshape(1, V))
    return logits.reshape(B, 1, V)


# speedup vs baseline: 2.5424x; 1.1649x over previous
import functools

import jax
import jax.numpy as jnp
from jax import lax
from jax.experimental import pallas as pl
from jax.experimental.pallas import tpu as pltpu
from jax.experimental.pallas import tpu_sc as plsc

B = 32
D = 512
V = 50257

_ROWS_PER_WORKER = 8
_ACTIVE_WORKERS = B // _ROWS_PER_WORKER

_sc_mesh = plsc.VectorSubcoreMesh(core_axis_name="c", subcore_axis_name="s")


@functools.partial(
    pl.kernel,
    out_type=jax.ShapeDtypeStruct((B, D), jnp.float32),
    mesh=_sc_mesh,
    scratch_types=[
        pltpu.VMEM((_ROWS_PER_WORKER,), jnp.int32),
        pltpu.VMEM((_ROWS_PER_WORKER, D), jnp.float32),
        pltpu.SemaphoreType.DMA,
    ],
)
def _sc_gather(emb_hbm, ids_hbm, out_hbm, idx_v, rows_v, sem):
    info = plsc.get_sparse_core_info()
    nc = info.num_cores
    wid = lax.axis_index("s") * nc + lax.axis_index("c")

    @pl.when(wid < _ACTIVE_WORKERS)
    def _():
        base = wid * _ROWS_PER_WORKER
        pltpu.sync_copy(ids_hbm.at[pl.ds(base, _ROWS_PER_WORKER)], idx_v)
        pltpu.async_copy(emb_hbm.at[idx_v], rows_v, sem).wait()
        pltpu.sync_copy(rows_v, out_hbm.at[pl.ds(base, _ROWS_PER_WORKER)])


_VB = 2048
_NV = (V + _VB - 1) // _VB


def _mm_body(x_ref, wt_ref, b_ref, o_ref):
    res = (
        lax.dot_general(
            x_ref[...],
            wt_ref[...],
            dimension_numbers=(((1,), (1,)), ((), ())),
            preferred_element_type=jnp.float32,
        )
        + b_ref[...]
    )
    o_ref[...] = res[:, None, :]


def _tc_project(x, WT, b2d):
    return pl.pallas_call(
        _mm_body,
        grid=(_NV,),
        in_specs=[
            pl.BlockSpec((B, D), lambda v: (0, 0)),
            pl.BlockSpec((_VB, D), lambda v: (v, 0)),
            pl.BlockSpec((1, _VB), lambda v: (0, v)),
        ],
        out_specs=pl.BlockSpec((B, 1, _VB), lambda v: (0, 0, v)),
        out_shape=jax.ShapeDtypeStruct((B, 1, V), jnp.float32),
        compiler_params=pltpu.CompilerParams(
            dimension_semantics=("arbitrary",),
        ),
    )(x, WT, b2d)


def kernel(input_ids, emb, W, b):
    ids = input_ids.reshape(B).astype(jnp.int32)
    x = _sc_gather(emb, ids)
    return _tc_project(x, W.T, b.reshape(1, V))


# speedup vs baseline: 3.5614x; 1.4008x over previous
import functools

import jax
import jax.numpy as jnp
from jax import lax
from jax.experimental import pallas as pl
from jax.experimental.pallas import tpu as pltpu

B = 32
D = 512
V = 50257

_VB = 2048
_NV = (V + _VB - 1) // _VB


def _mm_body(ids_ref, emb_hbm, wt_ref, b_ref, o_ref, x_vmem, sem):
    v = pl.program_id(0)

    @pl.when(v == 0)
    def _():
        for i in range(B):
            pltpu.make_async_copy(
                emb_hbm.at[pl.ds(ids_ref[i], 1)], x_vmem.at[pl.ds(i, 1)], sem
            ).start()
        for i in range(B):
            pltpu.make_async_copy(
                emb_hbm.at[pl.ds(ids_ref[i], 1)], x_vmem.at[pl.ds(i, 1)], sem
            ).wait()

    res = (
        lax.dot_general(
            x_vmem[...],
            wt_ref[...],
            dimension_numbers=(((1,), (1,)), ((), ())),
            preferred_element_type=jnp.float32,
        )
        + b_ref[...]
    )
    o_ref[...] = res[:, None, :]


def kernel(input_ids, emb, W, b):
    ids = input_ids.reshape(B).astype(jnp.int32)
    grid_spec = pltpu.PrefetchScalarGridSpec(
        num_scalar_prefetch=1,
        grid=(_NV,),
        in_specs=[
            pl.BlockSpec(memory_space=pltpu.MemorySpace.HBM),
            pl.BlockSpec((_VB, D), lambda v, ids: (v, 0)),
            pl.BlockSpec((1, _VB), lambda v, ids: (0, v)),
        ],
        out_specs=pl.BlockSpec((B, 1, _VB), lambda v, ids: (0, 0, v)),
        scratch_shapes=[
            pltpu.VMEM((B, D), jnp.float32),
            pltpu.SemaphoreType.DMA,
        ],
    )
    out = pl.pallas_call(
        _mm_body,
        grid_spec=grid_spec,
        out_shape=jax.ShapeDtypeStruct((B, 1, V), jnp.float32),
        compiler_params=pltpu.CompilerParams(
            dimension_semantics=("arbitrary",),
        ),
    )(ids, emb, W.T, b.reshape(1, V))
    return out


# speedup vs baseline: 3.9486x; 1.1087x over previous
import functools

import jax
import jax.numpy as jnp
from jax import lax
from jax.experimental import pallas as pl
from jax.experimental.pallas import tpu as pltpu

B = 32
D = 512
V = 50257

_VB = 4096
_NV = (V + _VB - 1) // _VB


def _mm_body(ids_ref, emb_hbm, wt_ref, b_ref, o_ref, x_vmem, sem):
    v = pl.program_id(0)

    @pl.when(v == 0)
    def _():
        for i in range(B):
            pltpu.make_async_copy(
                emb_hbm.at[pl.ds(ids_ref[i], 1)], x_vmem.at[pl.ds(i, 1)], sem
            ).start()
        for i in range(B):
            pltpu.make_async_copy(
                emb_hbm.at[pl.ds(ids_ref[i], 1)], x_vmem.at[pl.ds(i, 1)], sem
            ).wait()

    res = (
        lax.dot_general(
            x_vmem[...],
            wt_ref[...],
            dimension_numbers=(((1,), (1,)), ((), ())),
            preferred_element_type=jnp.float32,
        )
        + b_ref[...]
    )
    o_ref[...] = res[:, None, :]


def kernel(input_ids, emb, W, b):
    ids = input_ids.reshape(B).astype(jnp.int32)
    grid_spec = pltpu.PrefetchScalarGridSpec(
        num_scalar_prefetch=1,
        grid=(_NV,),
        in_specs=[
            pl.BlockSpec(memory_space=pltpu.MemorySpace.HBM),
            pl.BlockSpec((_VB, D), lambda v, ids: (v, 0)),
            pl.BlockSpec((1, _VB), lambda v, ids: (0, v)),
        ],
        out_specs=pl.BlockSpec((B, 1, _VB), lambda v, ids: (0, 0, v)),
        scratch_shapes=[
            pltpu.VMEM((B, D), jnp.float32),
            pltpu.SemaphoreType.DMA,
        ],
    )
    out = pl.pallas_call(
        _mm_body,
        grid_spec=grid_spec,
        out_shape=jax.ShapeDtypeStruct((B, 1, V), jnp.float32),
        compiler_params=pltpu.CompilerParams(
            dimension_semantics=("arbitrary",),
        ),
    )(ids, emb, W.T, b.reshape(1, V))
    return out
